# baseline (speedup 1.0000x reference)
import functools

import jax
import jax.numpy as jnp
from jax import lax
from jax.experimental import pallas as pl
from jax.experimental.pallas import tpu as pltpu
from jax.experimental.pallas import tpu_sc as plsc

NC = 2
NS = 16
L = 16
NW = NC * NS

_SC_PARAMS = pltpu.CompilerParams(
    use_tc_tiling_on_sc=True, needs_layout_passes=False,
    disable_bounds_checks=True)


def _make_relayout(V, D):
  assert D == 64
  NB = V // (2 * D)
  REM = V - NB * 2 * D
  per_tile = NB // NW
  extra = NB - per_tile * NW
  W = 2 * D

  mesh = plsc.VectorSubcoreMesh(core_axis_name="c", subcore_axis_name="s")

  @functools.partial(
      pl.kernel,
      mesh=mesh,
      out_type=jax.ShapeDtypeStruct((V // 2, W), jnp.float32),
      compiler_params=_SC_PARAMS,
      scratch_types=[
          pltpu.VMEM((D, W), jnp.float32),
          pltpu.VMEM((D, W), jnp.float32),
          pltpu.VMEM((D, W), jnp.float32),
          pltpu.VMEM((D, W), jnp.float32),
          pltpu.SemaphoreType.DMA,
          pltpu.SemaphoreType.DMA,
      ],
  )
  def relayout(tabT_hbm, rem_hbm, pairs_hbm, in0, in1, ob0, ob1, gsem,
               wsem):
    cid = lax.axis_index("c")
    sid = lax.axis_index("s")
    wid = cid * NS + sid
    nblk = per_tile + jnp.where(wid < extra, 1, 0)

    ins = (in0, in1)
    obs = (ob0, ob1)
    lanes = lax.iota(jnp.int32, L)

    def blk_of(i):
      return wid + i * NW

    def start_in(i, p):
      pltpu.async_copy(
          tabT_hbm.at[:, pl.ds(blk_of(i) * W, W)], ins[p], gsem)

    def wait_in(p):
      pltpu.make_async_copy(
          tabT_hbm.at[:, pl.ds(0, W)], ins[p], gsem).wait()

    def transpose(src, dst):
      def t_body(t, carry):
        c0 = jnp.full((L,), 0, jnp.int32) + (2 * t)
        c1 = c0 + 1
        for h in range(8):
          dvec = lanes + ((h % 4) * L)
          cvec = c0 if h < 4 else c1
          dst[t, pl.ds(h * L, L)] = plsc.load_gather(src, [dvec, cvec])
        return carry

      lax.fori_loop(0, D, t_body, 0)

    def write_out(i, p):
      pltpu.async_copy(
          obs[p], pairs_hbm.at[pl.ds(blk_of(i) * D, D)], wsem)

    def wait_out(p):
      pltpu.make_async_copy(
          obs[p], pairs_hbm.at[pl.ds(0, D)], wsem).wait()

    start_in(0, 0)

    def body(i, carry):
      p = lax.rem(i, 2)
      for q in range(2):

        @pl.when(p == q)
        def _():
          wait_in(q)

          @pl.when(i + 1 < nblk)
          def _():
            start_in(i + 1, 1 - q)

          @pl.when(i >= 2)
          def _():
            wait_out(q)

          transpose(ins[q], obs[q])
          write_out(i, q)
      return carry

    lax.fori_loop(0, nblk, body, 0)
    wait_out(0)
    wait_out(1)

    if REM:
      @pl.when(wid == extra)
      def _():
        pltpu.sync_copy(rem_hbm, in0.at[pl.ds(0, REM // 2)])
        pltpu.sync_copy(in0.at[pl.ds(0, REM // 2)],
                        pairs_hbm.at[pl.ds(NB * D, REM // 2)])

  return relayout


def _make_lookup_mean(V, D, S, B):
  assert B % NW == 0 and D == 64 and S % 2 == 0
  b_per_w = B // NW
  assert b_per_w == 128
  W = 2 * D

  mesh = plsc.VectorSubcoreMesh(core_axis_name="c", subcore_axis_name="s")

  @functools.partial(
      pl.kernel,
      mesh=mesh,
      out_type=jax.ShapeDtypeStruct((B, D), jnp.float32),
      compiler_params=_SC_PARAMS,
      scratch_types=[
          pltpu.VMEM((S, b_per_w), jnp.int32),
          pltpu.VMEM((S, b_per_w), jnp.int32),
          pltpu.VMEM((b_per_w, W), jnp.float32),
          pltpu.VMEM((b_per_w, W), jnp.float32),
          pltpu.VMEM((D, b_per_w), jnp.float32),
          pltpu.VMEM((b_per_w, D), jnp.float32),
          pltpu.SemaphoreType.DMA,
          pltpu.SemaphoreType.DMA,
      ],
  )
  def lookup_mean(pairs_hbm, idx_hbm, out_hbm, qidx_v, pcol_v, rows0,
                  rows1, accT, out_v, sem0, sem1):
    cid = lax.axis_index("c")
    sid = lax.axis_index("s")
    wid = cid * NS + sid
    base_glob = wid * b_per_w

    pltpu.sync_copy(idx_hbm.at[:, pl.ds(base_glob, b_per_w)], qidx_v)

    def split_body(s, carry):
      for c in range(b_per_w // L):
        v = qidx_v[s, pl.ds(c * L, L)]
        qidx_v[s, pl.ds(c * L, L)] = lax.shift_right_logical(v, 1)
        pcol_v[s, pl.ds(c * L, L)] = lax.shift_left(
            lax.bitwise_and(v, 1), 6)
      return carry

    lax.fori_loop(0, S, split_body, 0)

    zero = jnp.zeros((L,), jnp.float32)

    def zero_body(d, carry):
      for c in range(b_per_w // L):
        accT[d, pl.ds(c * L, L)] = zero
      return carry

    lax.fori_loop(0, D, zero_body, 0)

    bufs = (rows0, rows1)
    sems = (sem0, sem1)

    def start(s, p):
      pltpu.async_copy(pairs_hbm.at[qidx_v.at[s]], bufs[p], sems[p])

    def wait(p):
      pltpu.make_async_copy(
          pairs_hbm.at[qidx_v.at[0]], bufs[p], sems[p]).wait()

    lanes = lax.iota(jnp.int32, L)

    def accumulate(s, buf):
      for jc in range(b_per_w // L):
        jvec = lanes + (jc * L)
        col0 = pcol_v[s, pl.ds(jc * L, L)]

        def d_body(d16, col):
          base_d = d16 * L
          cols = []
          for k in range(L):
            cols.append(col)
            col = col + 1
          vals = [plsc.load_gather(buf, [jvec, c]) for c in cols]
          for k in range(L):
            plsc.addupdate(accT.at[base_d + k, pl.ds(jc * L, L)], vals[k])
          return col

        lax.fori_loop(0, D // L, d_body, col0)

    start(0, 0)
    start(1, 1)

    def body(k, carry):
      s = 2 * k
      for p in range(2):
        wait(p)
        accumulate(s + p, bufs[p])

        @pl.when(s + p + 2 < S)
        def _():
          start(s + p + 2, p)
      return carry

    lax.fori_loop(0, S // 2, body, 0)

    inv = jnp.float32(1.0 / S)

    def wb_body(j, carry):
      jsplat = jnp.full((L,), 0, jnp.int32) + j
      for k in range(D // L):
        vals = plsc.load_gather(accT, [lanes + (k * L), jsplat])
        out_v[j, pl.ds(k * L, L)] = vals * inv
      return carry

    lax.fori_loop(0, b_per_w, wb_body, 0)
    pltpu.sync_copy(out_v, out_hbm.at[pl.ds(base_glob, b_per_w)])

  return lookup_mean


def kernel(embedding_weight, input_seqs):
  V, D = embedding_weight.shape
  S, B = input_seqs.shape
  idx = input_seqs.astype(jnp.int32)
  tabT = embedding_weight.T
  NB = V // (2 * D)
  REM = V - NB * 2 * D
  rem = embedding_weight[V - REM:, :].reshape(REM // 2, 2 * D)
  pairs = _make_relayout(V, D)(tabT, rem)
  return _make_lookup_mean(V, D, S, B)(pairs, idx)


# speedup vs baseline: 1.2047x; 1.2047x over previous
import functools

import jax
import jax.numpy as jnp
from jax import lax
from jax.experimental import pallas as pl
from jax.experimental.pallas import tpu as pltpu
from jax.experimental.pallas import tpu_sc as plsc

NC = 2
NS = 16
L = 16
NW = NC * NS

_SC_PARAMS = pltpu.CompilerParams(
    use_tc_tiling_on_sc=True, needs_layout_passes=False,
    disable_bounds_checks=True)


def _make_relayout(V, D):
  assert D == 64
  NB = V // (2 * D)
  REM = V - NB * 2 * D
  per_tile = NB // NW
  extra = NB - per_tile * NW
  W = 2 * D

  mesh = plsc.VectorSubcoreMesh(core_axis_name="c", subcore_axis_name="s")

  @functools.partial(
      pl.kernel,
      mesh=mesh,
      out_type=jax.ShapeDtypeStruct((V // 2, W), jnp.float32),
      compiler_params=_SC_PARAMS,
      scratch_types=[
          pltpu.VMEM((D, W), jnp.float32),
          pltpu.VMEM((D, W), jnp.float32),
          pltpu.VMEM((D, W), jnp.float32),
          pltpu.VMEM((D, W), jnp.float32),
          pltpu.SemaphoreType.DMA,
          pltpu.SemaphoreType.DMA,
      ],
  )
  def relayout(tabT_hbm, rem_hbm, pairs_hbm, in0, in1, ob0, ob1, gsem,
               wsem):
    cid = lax.axis_index("c")
    sid = lax.axis_index("s")
    wid = cid * NS + sid
    nblk = per_tile + jnp.where(wid < extra, 1, 0)

    ins = (in0, in1)
    obs = (ob0, ob1)
    lanes = lax.iota(jnp.int32, L)

    def blk_of(i):
      return wid + i * NW

    def start_in(i, p):
      for g in range(D // 8):
        pltpu.async_copy(
            tabT_hbm.at[pl.ds(g * 8, 8), pl.ds(blk_of(i) * W, W)],
            ins[p].at[pl.ds(g * 8, 8)], gsem)

    def wait_in(p):
      for g in range(D // 8):
        pltpu.make_async_copy(
            tabT_hbm.at[pl.ds(0, 8), pl.ds(0, W)],
            ins[p].at[pl.ds(g * 8, 8)], gsem).wait()

    def transpose(src, dst):
      def t_body(t4, carry):
        vals = []
        for u in range(4):
          t = t4 * 4 + u
          c0 = jnp.full((L,), 0, jnp.int32) + (2 * t)
          c1 = c0 + 1
          for h in range(8):
            dvec = lanes + ((h % 4) * L)
            cvec = c0 if h < 4 else c1
            vals.append((t, h, plsc.load_gather(src, [dvec, cvec])))
        for t, h, v in vals:
          dst[t, pl.ds(h * L, L)] = v
        return carry

      lax.fori_loop(0, D // 4, t_body, 0)

    def write_out(i, p):
      pltpu.async_copy(
          obs[p], pairs_hbm.at[pl.ds(blk_of(i) * D, D)], wsem)

    def wait_out(p):
      pltpu.make_async_copy(
          obs[p], pairs_hbm.at[pl.ds(0, D)], wsem).wait()

    start_in(0, 0)

    def body(i, carry):
      p = lax.rem(i, 2)
      for q in range(2):

        @pl.when(p == q)
        def _():
          wait_in(q)

          @pl.when(i + 1 < nblk)
          def _():
            start_in(i + 1, 1 - q)

          @pl.when(i >= 2)
          def _():
            wait_out(q)

          transpose(ins[q], obs[q])
          write_out(i, q)
      return carry

    lax.fori_loop(0, nblk, body, 0)
    wait_out(0)
    wait_out(1)

    if REM:
      @pl.when(wid == extra)
      def _():
        pltpu.sync_copy(rem_hbm, in0.at[pl.ds(0, REM // 2)])
        pltpu.sync_copy(in0.at[pl.ds(0, REM // 2)],
                        pairs_hbm.at[pl.ds(NB * D, REM // 2)])

  return relayout


def _make_lookup_mean(V, D, S, B):
  assert B % NW == 0 and D == 64 and S % 4 == 0
  b_per_w = B // NW
  assert b_per_w == 128
  W = 2 * D
  NBUF = 4

  mesh = plsc.VectorSubcoreMesh(core_axis_name="c", subcore_axis_name="s")

  @functools.partial(
      pl.kernel,
      mesh=mesh,
      out_type=jax.ShapeDtypeStruct((B, D), jnp.float32),
      compiler_params=_SC_PARAMS,
      scratch_types=[
          pltpu.VMEM((S, b_per_w), jnp.int32),
          pltpu.VMEM((NBUF, b_per_w), jnp.int32),
          pltpu.VMEM((b_per_w, W), jnp.float32),
          pltpu.VMEM((b_per_w, W), jnp.float32),
          pltpu.VMEM((b_per_w, W), jnp.float32),
          pltpu.VMEM((b_per_w, W), jnp.float32),
          pltpu.VMEM((D, b_per_w), jnp.float32),
          pltpu.VMEM((b_per_w, D), jnp.float32),
          pltpu.SemaphoreType.DMA,
          pltpu.SemaphoreType.DMA,
          pltpu.SemaphoreType.DMA,
          pltpu.SemaphoreType.DMA,
      ],
  )
  def lookup_mean(pairs_hbm, idx_hbm, out_hbm, idx_v, qrows, rows0,
                  rows1, rows2, rows3, accT, out_v, sem0, sem1, sem2,
                  sem3):
    cid = lax.axis_index("c")
    sid = lax.axis_index("s")
    wid = cid * NS + sid
    base_glob = wid * b_per_w

    pltpu.sync_copy(idx_hbm.at[:, pl.ds(base_glob, b_per_w)], idx_v)

    zero = jnp.zeros((L,), jnp.float32)

    def zero_body(d, carry):
      for c in range(b_per_w // L):
        accT[d, pl.ds(c * L, L)] = zero
      return carry

    lax.fori_loop(0, D, zero_body, 0)

    bufs = (rows0, rows1, rows2, rows3)
    sems = (sem0, sem1, sem2, sem3)
    lanes = lax.iota(jnp.int32, L)

    def start(s, p):
      for c in range(b_per_w // L):
        qrows[p, pl.ds(c * L, L)] = lax.shift_right_logical(
            idx_v[s, pl.ds(c * L, L)], 1)
      pltpu.async_copy(pairs_hbm.at[qrows.at[p]], bufs[p], sems[p])

    def wait(p):
      pltpu.make_async_copy(
          pairs_hbm.at[qrows.at[p]], bufs[p], sems[p]).wait()

    def accumulate(s, buf):
      for jc in range(b_per_w // L):
        jvec = lanes + (jc * L)
        col0 = lax.shift_left(
            lax.bitwise_and(idx_v[s, pl.ds(jc * L, L)], 1), 6)

        def d_body(d16, col):
          base_d = d16 * L
          cols = []
          for k in range(L):
            cols.append(col)
            col = col + 1
          vals = [plsc.load_gather(buf, [jvec, c]) for c in cols]
          for k in range(L):
            plsc.addupdate(accT.at[base_d + k, pl.ds(jc * L, L)], vals[k])
          return col

        lax.fori_loop(0, D // L, d_body, col0)

    for p in range(NBUF):
      start(p, p)

    def body(k, carry):
      s = NBUF * k
      for p in range(NBUF):
        wait(p)
        accumulate(s + p, bufs[p])

        @pl.when(s + p + NBUF < S)
        def _():
          start(s + p + NBUF, p)
      return carry

    lax.fori_loop(0, S // NBUF, body, 0)

    inv = jnp.float32(1.0 / S)

    def wb_body(j, carry):
      jsplat = jnp.full((L,), 0, jnp.int32) + j
      for k in range(D // L):
        vals = plsc.load_gather(accT, [lanes + (k * L), jsplat])
        out_v[j, pl.ds(k * L, L)] = vals * inv
      return carry

    lax.fori_loop(0, b_per_w, wb_body, 0)
    pltpu.sync_copy(out_v, out_hbm.at[pl.ds(base_glob, b_per_w)])

  return lookup_mean


def kernel(embedding_weight, input_seqs):
  V, D = embedding_weight.shape
  S, B = input_seqs.shape
  idx = input_seqs.astype(jnp.int32)
  tabT = embedding_weight.T
  NB = V // (2 * D)
  REM = V - NB * 2 * D
  rem = embedding_weight[V - REM:, :].reshape(REM // 2, 2 * D)
  pairs = _make_relayout(V, D)(tabT, rem)
  return _make_lookup_mean(V, D, S, B)(pairs, idx)


# speedup vs baseline: 4.1210x; 3.4209x over previous
import functools

import jax
import jax.numpy as jnp
from jax import lax
from jax.experimental import pallas as pl
from jax.experimental.pallas import tpu as pltpu
from jax.experimental.pallas import tpu_sc as plsc

NC = 2
NS = 16
L = 16
NW = NC * NS

_TC_BLK = 2048


def _make_relayout(V, D):
  grid = (V + 2 * _TC_BLK - 1) // (2 * _TC_BLK)
  nchunk = V // _TC_BLK

  def relayout_block(even_ref, odd_ref, tail_ref, out_ref):
    i = pl.program_id(0)

    @pl.when(i < grid - 1)
    def _():
      out_ref[...] = jnp.concatenate(
          [even_ref[...].T, odd_ref[...].T], axis=1)

    @pl.when(i == grid - 1)
    def _():
      t = tail_ref[...].T
      out_ref[...] = jnp.concatenate([t, t], axis=1)

  return pl.pallas_call(
      relayout_block,
      grid=(grid,),
      in_specs=[
          pl.BlockSpec((D, _TC_BLK),
                       lambda i: (0, jnp.minimum(2 * i, nchunk - 2))),
          pl.BlockSpec((D, _TC_BLK),
                       lambda i: (0, jnp.minimum(2 * i + 1, nchunk - 1))),
          pl.BlockSpec((D, _TC_BLK), lambda i: (0, 0)),
      ],
      out_specs=pl.BlockSpec((_TC_BLK, 2 * D), lambda i: (i, 0)),
      out_shape=jax.ShapeDtypeStruct((grid * _TC_BLK, 2 * D),
                                     jnp.float32),
  )


def _make_lookup_mean(VR, D, S, B):
  assert B % NW == 0
  b_per_w = B // NW
  assert b_per_w % 8 == 0

  mesh = plsc.VectorSubcoreMesh(core_axis_name="c", subcore_axis_name="s")

  @functools.partial(
      pl.kernel,
      mesh=mesh,
      out_type=jax.ShapeDtypeStruct((B, D), jnp.float32),
      compiler_params=pltpu.CompilerParams(use_tc_tiling_on_sc=False),
      scratch_types=[
          pltpu.VMEM((S, b_per_w), jnp.int32),
          pltpu.VMEM((b_per_w, D), jnp.float32),
          pltpu.VMEM((b_per_w, D), jnp.float32),
          pltpu.VMEM((b_per_w,), jnp.int32),
          pltpu.VMEM_SHARED((NS * b_per_w, D), jnp.float32),
          pltpu.SemaphoreType.DMA,
          pltpu.SemaphoreType.DMA,
      ],
  )
  def lookup_mean(table_hbm, idx_hbm, out_hbm, idx_v, rows0, rows1,
                  ramp_v, acc_sh, sem0, sem1):
    cid = lax.axis_index("c")
    sid = lax.axis_index("s")
    wid = cid * NS + sid
    base_local = sid * b_per_w
    base_glob = wid * b_per_w

    pltpu.sync_copy(idx_hbm.at[:, pl.ds(base_glob, b_per_w)], idx_v)

    def remap_body(s, carry):
      for c in range(b_per_w // L):
        v = idx_v[s, pl.ds(c * L, L)]
        r = (lax.bitwise_and(v, ~(2 * _TC_BLK - 1))
             + lax.shift_left(lax.bitwise_and(v, _TC_BLK - 1), 1)
             + lax.bitwise_and(lax.shift_right_logical(v, 11), 1))
        idx_v[s, pl.ds(c * L, L)] = r
      return carry

    lax.fori_loop(0, S, remap_body, 0)

    for i in range(b_per_w // L):
      ramp_v[pl.ds(i * L, L)] = (
          lax.iota(jnp.int32, L) + (base_local + i * L))

    bufs = (rows0, rows1)
    sems = (sem0, sem1)

    def start(s, p):
      pltpu.async_copy(table_hbm.at[idx_v.at[s]], bufs[p], sems[p])

    def wait(p):
      pltpu.make_async_copy(
          table_hbm.at[idx_v.at[0]], bufs[p], sems[p]).wait()

    start(0, 0)
    start(1, 1)
    wait(0)
    pltpu.sync_copy(rows0, acc_sh.at[ramp_v])
    wait(1)
    pltpu.sync_copy(rows1, acc_sh.at[ramp_v], add=True)

    def body(k, carry):
      s = 2 * k
      for p in range(2):
        start(s + p, p)
      for p in range(2):
        wait(p)
        pltpu.sync_copy(bufs[p], acc_sh.at[ramp_v], add=True)
      return carry

    lax.fori_loop(1, S // 2, body, 0)

    pltpu.sync_copy(acc_sh.at[pl.ds(base_local, b_per_w)], rows0)
    inv = jnp.float32(1.0 / S)

    def scale_body(b, carry):
      for c in range(D // L):
        rows0[b, pl.ds(c * L, L)] = rows0[b, pl.ds(c * L, L)] * inv
      return carry

    lax.fori_loop(0, b_per_w, scale_body, 0)
    pltpu.sync_copy(rows0, out_hbm.at[pl.ds(base_glob, b_per_w)])

  return lookup_mean


def kernel(embedding_weight, input_seqs):
  V, D = embedding_weight.shape
  S, B = input_seqs.shape
  idx = input_seqs.astype(jnp.int32)
  tabT = embedding_weight.T
  ntail = V - (V // (2 * _TC_BLK)) * 2 * _TC_BLK
  tail = jnp.pad(embedding_weight[V - ntail:, :],
                 ((0, _TC_BLK - ntail), (0, 0))).T
  pairs = _make_relayout(V, D)(tabT, tabT, tail)
  VR = pairs.shape[0] * 2
  table_lin = pairs.reshape(VR, D)
  return _make_lookup_mean(VR, D, S, B)(table_lin, idx)


# speedup vs baseline: 4.7199x; 1.1453x over previous
import functools

import jax
import jax.numpy as jnp
from jax import lax
from jax.experimental import pallas as pl
from jax.experimental.pallas import tpu as pltpu
from jax.experimental.pallas import tpu_sc as plsc

NC = 2
NS = 16
L = 16
NW = NC * NS

_TC_BLK = 4096
_TC_SHIFT = _TC_BLK.bit_length() - 1


def _make_relayout(V, D):
  grid = (V + 2 * _TC_BLK - 1) // (2 * _TC_BLK)
  nchunk = V // _TC_BLK

  def relayout_block(even_ref, odd_ref, tail_ref, out_ref):
    i = pl.program_id(0)

    @pl.when(i < grid - 1)
    def _():
      out_ref[...] = jnp.concatenate(
          [even_ref[...].T, odd_ref[...].T], axis=1)

    @pl.when(i == grid - 1)
    def _():
      t = tail_ref[...].T
      out_ref[...] = jnp.concatenate([t, t], axis=1)

  return pl.pallas_call(
      relayout_block,
      grid=(grid,),
      in_specs=[
          pl.BlockSpec((D, _TC_BLK),
                       lambda i: (0, jnp.minimum(2 * i, nchunk - 2))),
          pl.BlockSpec((D, _TC_BLK),
                       lambda i: (0, jnp.minimum(2 * i + 1, nchunk - 1))),
          pl.BlockSpec((D, _TC_BLK), lambda i: (0, 0)),
      ],
      out_specs=pl.BlockSpec((_TC_BLK, 2 * D), lambda i: (i, 0)),
      out_shape=jax.ShapeDtypeStruct((grid * _TC_BLK, 2 * D),
                                     jnp.float32),
  )


def _make_lookup_mean(VR, D, S, B):
  assert B % NW == 0
  b_per_w = B // NW
  assert b_per_w % 8 == 0

  mesh = plsc.VectorSubcoreMesh(core_axis_name="c", subcore_axis_name="s")

  @functools.partial(
      pl.kernel,
      mesh=mesh,
      out_type=jax.ShapeDtypeStruct((B, D), jnp.float32),
      compiler_params=pltpu.CompilerParams(use_tc_tiling_on_sc=False),
      scratch_types=[
          pltpu.VMEM((S, b_per_w), jnp.int32),
          pltpu.VMEM((b_per_w, D), jnp.float32),
          pltpu.VMEM((b_per_w, D), jnp.float32),
          pltpu.VMEM((b_per_w,), jnp.int32),
          pltpu.VMEM_SHARED((NS * b_per_w, D), jnp.float32),
          pltpu.SemaphoreType.DMA,
          pltpu.SemaphoreType.DMA,
      ],
  )
  def lookup_mean(table_hbm, idx_hbm, out_hbm, idx_v, rows0, rows1,
                  ramp_v, acc_sh, sem0, sem1):
    cid = lax.axis_index("c")
    sid = lax.axis_index("s")
    wid = cid * NS + sid
    base_local = sid * b_per_w
    base_glob = wid * b_per_w

    pltpu.sync_copy(idx_hbm.at[:, pl.ds(base_glob, b_per_w)], idx_v)

    def remap_body(s, carry):
      for c in range(b_per_w // L):
        v = idx_v[s, pl.ds(c * L, L)]
        r = (lax.bitwise_and(v, ~(2 * _TC_BLK - 1))
             + lax.shift_left(lax.bitwise_and(v, _TC_BLK - 1), 1)
             + lax.bitwise_and(lax.shift_right_logical(v, _TC_SHIFT), 1))
        idx_v[s, pl.ds(c * L, L)] = r
      return carry

    lax.fori_loop(0, S, remap_body, 0)

    for i in range(b_per_w // L):
      ramp_v[pl.ds(i * L, L)] = (
          lax.iota(jnp.int32, L) + (base_local + i * L))

    bufs = (rows0, rows1)
    sems = (sem0, sem1)

    def start(s, p):
      pltpu.async_copy(table_hbm.at[idx_v.at[s]], bufs[p], sems[p])

    def wait(p):
      pltpu.make_async_copy(
          table_hbm.at[idx_v.at[0]], bufs[p], sems[p]).wait()

    start(0, 0)
    start(1, 1)
    wait(0)
    pltpu.sync_copy(rows0, acc_sh.at[ramp_v])
    wait(1)
    pltpu.sync_copy(rows1, acc_sh.at[ramp_v], add=True)

    def body(k, carry):
      s = 2 * k
      for p in range(2):
        start(s + p, p)
      for p in range(2):
        wait(p)
        pltpu.sync_copy(bufs[p], acc_sh.at[ramp_v], add=True)
      return carry

    lax.fori_loop(1, S // 2, body, 0)

    pltpu.sync_copy(acc_sh.at[pl.ds(base_local, b_per_w)], rows0)
    inv = jnp.float32(1.0 / S)

    def scale_body(b, carry):
      for c in range(D // L):
        rows0[b, pl.ds(c * L, L)] = rows0[b, pl.ds(c * L, L)] * inv
      return carry

    lax.fori_loop(0, b_per_w, scale_body, 0)
    pltpu.sync_copy(rows0, out_hbm.at[pl.ds(base_glob, b_per_w)])

  return lookup_mean


def kernel(embedding_weight, input_seqs):
  V, D = embedding_weight.shape
  S, B = input_seqs.shape
  idx = input_seqs.astype(jnp.int32)
  tabT = embedding_weight.T
  ntail = V - (V // (2 * _TC_BLK)) * 2 * _TC_BLK
  tail = jnp.pad(embedding_weight[V - ntail:, :],
                 ((0, _TC_BLK - ntail), (0, 0))).T
  pairs = _make_relayout(V, D)(tabT, tabT, tail)
  VR = pairs.shape[0] * 2
  table_lin = pairs.reshape(VR, D)
  return _make_lookup_mean(VR, D, S, B)(table_lin, idx)


# speedup vs baseline: 5.0560x; 1.0712x over previous
import functools

import jax
import jax.numpy as jnp
from jax import lax
from jax.experimental import pallas as pl
from jax.experimental.pallas import tpu as pltpu
from jax.experimental.pallas import tpu_sc as plsc

NC = 2
NS = 16
L = 16
NW = NC * NS

_TC_BLK = 8192
_TC_SHIFT = _TC_BLK.bit_length() - 1


def _make_relayout(V, D):
  grid = (V + 2 * _TC_BLK - 1) // (2 * _TC_BLK)
  nchunk = V // _TC_BLK

  def relayout_block(even_ref, odd_ref, tail_ref, out_ref):
    i = pl.program_id(0)

    @pl.when(i < grid - 1)
    def _():
      out_ref[...] = jnp.concatenate(
          [even_ref[...].T, odd_ref[...].T], axis=1)

    @pl.when(i == grid - 1)
    def _():
      t = tail_ref[...].T
      out_ref[...] = jnp.concatenate([t, t], axis=1)

  return pl.pallas_call(
      relayout_block,
      grid=(grid,),
      in_specs=[
          pl.BlockSpec((D, _TC_BLK),
                       lambda i: (0, jnp.minimum(2 * i, nchunk - 2))),
          pl.BlockSpec((D, _TC_BLK),
                       lambda i: (0, jnp.minimum(2 * i + 1, nchunk - 1))),
          pl.BlockSpec((D, _TC_BLK), lambda i: (0, 0)),
      ],
      out_specs=pl.BlockSpec((_TC_BLK, 2 * D), lambda i: (i, 0)),
      out_shape=jax.ShapeDtypeStruct((grid * _TC_BLK, 2 * D),
                                     jnp.float32),
  )


def _make_lookup_mean(VR, D, S, B):
  assert B % NW == 0
  b_per_w = B // NW
  assert b_per_w % 8 == 0

  mesh = plsc.VectorSubcoreMesh(core_axis_name="c", subcore_axis_name="s")

  @functools.partial(
      pl.kernel,
      mesh=mesh,
      out_type=jax.ShapeDtypeStruct((B, D), jnp.float32),
      compiler_params=pltpu.CompilerParams(use_tc_tiling_on_sc=False),
      scratch_types=[
          pltpu.VMEM((S, b_per_w), jnp.int32),
          pltpu.VMEM((b_per_w, D), jnp.float32),
          pltpu.VMEM((b_per_w, D), jnp.float32),
          pltpu.VMEM((b_per_w,), jnp.int32),
          pltpu.VMEM_SHARED((NS * b_per_w, D), jnp.float32),
          pltpu.SemaphoreType.DMA,
          pltpu.SemaphoreType.DMA,
      ],
  )
  def lookup_mean(table_hbm, idx_hbm, out_hbm, idx_v, rows0, rows1,
                  ramp_v, acc_sh, sem0, sem1):
    cid = lax.axis_index("c")
    sid = lax.axis_index("s")
    wid = cid * NS + sid
    base_local = sid * b_per_w
    base_glob = wid * b_per_w

    pltpu.sync_copy(idx_hbm.at[:, pl.ds(base_glob, b_per_w)], idx_v)

    def remap_body(s, carry):
      for c in range(b_per_w // L):
        v = idx_v[s, pl.ds(c * L, L)]
        r = (lax.bitwise_and(v, ~(2 * _TC_BLK - 1))
             + lax.shift_left(lax.bitwise_and(v, _TC_BLK - 1), 1)
             + lax.bitwise_and(lax.shift_right_logical(v, _TC_SHIFT), 1))
        idx_v[s, pl.ds(c * L, L)] = r
      return carry

    lax.fori_loop(0, S, remap_body, 0)

    for i in range(b_per_w // L):
      ramp_v[pl.ds(i * L, L)] = (
          lax.iota(jnp.int32, L) + (base_local + i * L))

    bufs = (rows0, rows1)
    sems = (sem0, sem1)

    def start(s, p):
      pltpu.async_copy(table_hbm.at[idx_v.at[s]], bufs[p], sems[p])

    def wait(p):
      pltpu.make_async_copy(
          table_hbm.at[idx_v.at[0]], bufs[p], sems[p]).wait()

    start(0, 0)
    start(1, 1)
    wait(0)
    pltpu.sync_copy(rows0, acc_sh.at[ramp_v])
    wait(1)
    pltpu.sync_copy(rows1, acc_sh.at[ramp_v], add=True)

    def body(k, carry):
      s = 2 * k
      for p in range(2):
        start(s + p, p)
      for p in range(2):
        wait(p)
        pltpu.sync_copy(bufs[p], acc_sh.at[ramp_v], add=True)
      return carry

    lax.fori_loop(1, S // 2, body, 0)

    pltpu.sync_copy(acc_sh.at[pl.ds(base_local, b_per_w)], rows0)
    inv = jnp.float32(1.0 / S)

    def scale_body(b, carry):
      for c in range(D // L):
        rows0[b, pl.ds(c * L, L)] = rows0[b, pl.ds(c * L, L)] * inv
      return carry

    lax.fori_loop(0, b_per_w, scale_body, 0)
    pltpu.sync_copy(rows0, out_hbm.at[pl.ds(base_glob, b_per_w)])

  return lookup_mean


def kernel(embedding_weight, input_seqs):
  V, D = embedding_weight.shape
  S, B = input_seqs.shape
  idx = input_seqs.astype(jnp.int32)
  tabT = embedding_weight.T
  ntail = V - (V // (2 * _TC_BLK)) * 2 * _TC_BLK
  tail = jnp.pad(embedding_weight[V - ntail:, :],
                 ((0, _TC_BLK - ntail), (0, 0))).T
  pairs = _make_relayout(V, D)(tabT, tabT, tail)
  VR = pairs.shape[0] * 2
  table_lin = pairs.reshape(VR, D)
  return _make_lookup_mean(VR, D, S, B)(table_lin, idx)
